# Initial kernel scaffold; baseline (speedup 1.0000x reference)
import functools

import jax
import jax.numpy as jnp
from jax import lax
from jax.experimental import pallas as pl
from jax.experimental.pallas import tpu as pltpu
from jax.experimental.pallas import tpu_sc as plsc

N = 10000
E = 320000
D = 128
NC = 2
NS = 16
NW = NC * NS
G = 128
NGROUPS = E // G
ROWS_PER_TILE = N // NS

_mesh = plsc.VectorSubcoreMesh(core_axis_name="c", subcore_axis_name="s")


@functools.partial(
    pl.kernel,
    out_type=jax.ShapeDtypeStruct((NW, N), jnp.float32),
    mesh=_mesh,
    scratch_types=[
        pltpu.VMEM((G,), jnp.int32),
        pltpu.VMEM((G,), jnp.float32),
        pltpu.VMEM((N,), jnp.float32),
    ],
)
def _deg_kernel(col_hbm, ew_hbm, degp_hbm, colbuf, ewbuf, deg_local):
    c = lax.axis_index("c")
    s = lax.axis_index("s")
    wid = s * NC + c

    zero16 = jnp.zeros((16,), jnp.float32)

    def zbody(i, carry):
        deg_local[pl.ds(i * 16, 16)] = zero16
        return carry

    lax.fori_loop(0, N // 16, zbody, 0)

    lo = wid * NGROUPS // NW
    hi = (wid + 1) * NGROUPS // NW

    def gbody(g, carry):
        pltpu.sync_copy(col_hbm.at[pl.ds(g * G, G)], colbuf)
        pltpu.sync_copy(ew_hbm.at[pl.ds(g * G, G)], ewbuf)

        def inner(k, carry2):
            cv = colbuf[pl.ds(k * 16, 16)]
            wv = ewbuf[pl.ds(k * 16, 16)]
            plsc.addupdate_scatter(deg_local, [cv], wv)
            return carry2

        lax.fori_loop(0, G // 16, inner, 0)
        return carry

    lax.fori_loop(lo, hi, gbody, 0)
    pltpu.sync_copy(deg_local, degp_hbm.at[wid])


@functools.partial(
    pl.kernel,
    out_type=jax.ShapeDtypeStruct((NC, N, D), jnp.float32),
    mesh=_mesh,
    scratch_types=[
        pltpu.VMEM((G,), jnp.int32),
        pltpu.VMEM((G,), jnp.int32),
        pltpu.VMEM((G,), jnp.float32),
        pltpu.VMEM((G, D), jnp.float32),
        pltpu.VMEM_SHARED((N, D), jnp.float32),
        pltpu.SemaphoreType.DMA,
    ],
)
def _layer_kernel(row_hbm, col_hbm, ew_hbm, h_hbm, zeros_hbm, outp_hbm,
                  rowidx, colidx, ewbuf, rows, acc, dma_sem):
    c = lax.axis_index("c")
    s = lax.axis_index("s")
    wid = s * NC + c

    pltpu.sync_copy(zeros_hbm.at[pl.ds(s * ROWS_PER_TILE, ROWS_PER_TILE)],
                    acc.at[pl.ds(s * ROWS_PER_TILE, ROWS_PER_TILE)])
    plsc.subcore_barrier()

    lo = wid * NGROUPS // NW
    hi = (wid + 1) * NGROUPS // NW

    def gbody(g, carry):
        base = g * G
        pltpu.sync_copy(row_hbm.at[pl.ds(base, G)], rowidx)
        pltpu.sync_copy(col_hbm.at[pl.ds(base, G)], colidx)
        pltpu.sync_copy(ew_hbm.at[pl.ds(base, G)], ewbuf)
        pltpu.async_copy(h_hbm.at[rowidx], rows, dma_sem).wait()

        def scale(k, carry2):
            e0 = k * 16
            for j in range(16):
                w16 = plsc.load_gather(
                    ewbuf, [jnp.full((16,), e0 + j, jnp.int32)])
                for f in range(D // 16):
                    sl = pl.ds(f * 16, 16)
                    rows[e0 + j, sl] = rows[e0 + j, sl] * w16
            return carry2

        lax.fori_loop(0, G // 16, scale, 0)

        pltpu.sync_copy(rows, acc.at[colidx], add=True)
        return carry

    lax.fori_loop(lo, hi, gbody, 0)
    plsc.subcore_barrier()

    pltpu.sync_copy(acc.at[pl.ds(s * ROWS_PER_TILE, ROWS_PER_TILE)],
                    outp_hbm.at[c, pl.ds(s * ROWS_PER_TILE, ROWS_PER_TILE)])


def _dis_body(degp_ref, dis_ref):
    deg = jnp.sum(degp_ref[...], axis=0, keepdims=True) + 1.0
    safe = jnp.where(deg > 0, deg, 1.0)
    dis_ref[...] = jnp.where(deg > 0, lax.rsqrt(safe), 0.0)


def _dis_call(degp):
    return pl.pallas_call(
        _dis_body,
        out_shape=jax.ShapeDtypeStruct((1, N), jnp.float32),
    )(degp)


_BLK = 2000
_NBLK = N // _BLK


def _mm_body(x_ref, w_ref, disc_ref, out_ref):
    h = lax.dot_general(x_ref[...], w_ref[...], (((1,), (1,)), ((), ())),
                        preferred_element_type=jnp.float32)
    out_ref[...] = h * disc_ref[...]


def _mm_call(x, w, disc):
    return pl.pallas_call(
        _mm_body,
        grid=(_NBLK,),
        in_specs=[
            pl.BlockSpec((_BLK, D), lambda i: (i, 0)),
            pl.BlockSpec((D, D), lambda i: (0, 0)),
            pl.BlockSpec((_BLK, 1), lambda i: (i, 0)),
        ],
        out_specs=pl.BlockSpec((_BLK, D), lambda i: (i, 0)),
        out_shape=jax.ShapeDtypeStruct((N, D), jnp.float32),
    )(x, w, disc)


def _mid_body(p_ref, hp_ref, disc_ref, b_ref, w_ref, out_ref):
    srow = p_ref[0] + p_ref[1] + hp_ref[...]
    z = jnp.maximum(disc_ref[...] * srow + b_ref[...], 0.0)
    h2 = lax.dot_general(z, w_ref[...], (((1,), (1,)), ((), ())),
                         preferred_element_type=jnp.float32)
    out_ref[...] = h2 * disc_ref[...]


def _mid_call(p, hp, disc, b, w):
    return pl.pallas_call(
        _mid_body,
        grid=(_NBLK,),
        in_specs=[
            pl.BlockSpec((NC, _BLK, D), lambda i: (0, i, 0)),
            pl.BlockSpec((_BLK, D), lambda i: (i, 0)),
            pl.BlockSpec((_BLK, 1), lambda i: (i, 0)),
            pl.BlockSpec((1, D), lambda i: (0, 0)),
            pl.BlockSpec((D, D), lambda i: (0, 0)),
        ],
        out_specs=pl.BlockSpec((_BLK, D), lambda i: (i, 0)),
        out_shape=jax.ShapeDtypeStruct((N, D), jnp.float32),
    )(p, hp, disc, b, w)


def _final_body(p_ref, hp_ref, disc_ref, b_ref, out_ref):
    srow = p_ref[0] + p_ref[1] + hp_ref[...]
    out_ref[...] = jnp.maximum(disc_ref[...] * srow + b_ref[...], 0.0)


def _final_call(p, hp, disc, b):
    return pl.pallas_call(
        _final_body,
        grid=(_NBLK,),
        in_specs=[
            pl.BlockSpec((NC, _BLK, D), lambda i: (0, i, 0)),
            pl.BlockSpec((_BLK, D), lambda i: (i, 0)),
            pl.BlockSpec((_BLK, 1), lambda i: (i, 0)),
            pl.BlockSpec((1, D), lambda i: (0, 0)),
        ],
        out_specs=pl.BlockSpec((_BLK, D), lambda i: (i, 0)),
        out_shape=jax.ShapeDtypeStruct((N, D), jnp.float32),
    )(p, hp, disc, b)


def kernel(x, edge_index, edge_weights, W1, b1, W2, b2):
    row = edge_index[0]
    col = edge_index[1]
    zeros_nd = jnp.zeros((N, D), jnp.float32)

    degp = _deg_kernel(col, edge_weights)
    dis = _dis_call(degp)
    disc = dis.reshape(N, 1)

    h1p = _mm_call(x, W1, disc)
    p1 = _layer_kernel(row, col, edge_weights, h1p, zeros_nd)
    h2p = _mid_call(p1, h1p, disc, b1.reshape(1, D), W2)
    p2 = _layer_kernel(row, col, edge_weights, h2p, zeros_nd)
    return _final_call(p2, h2p, disc, b2.reshape(1, D))


# speedup vs baseline: 11.3973x; 11.3973x over previous
import functools

import jax
import jax.numpy as jnp
from jax import lax
from jax.experimental import pallas as pl
from jax.experimental.pallas import tpu as pltpu
from jax.experimental.pallas import tpu_sc as plsc

N = 10000
E = 320000
D = 128
NC = 2
NS = 16
NW = NC * NS
G = 128
NGROUPS = E // G
ROWS_PER_TILE = N // NS

_mesh = plsc.VectorSubcoreMesh(core_axis_name="c", subcore_axis_name="s")


@functools.partial(
    pl.kernel,
    out_type=jax.ShapeDtypeStruct((NW, N), jnp.float32),
    mesh=_mesh,
    scratch_types=[
        pltpu.VMEM((G,), jnp.int32),
        pltpu.VMEM((G,), jnp.float32),
        pltpu.VMEM((N,), jnp.float32),
    ],
    compiler_params=pltpu.CompilerParams(needs_layout_passes=False, use_tc_tiling_on_sc=False),
)
def _deg_kernel(col_hbm, ew_hbm, degp_hbm, colbuf, ewbuf, deg_local):
    c = lax.axis_index("c")
    s = lax.axis_index("s")
    wid = s * NC + c

    zero16 = jnp.zeros((16,), jnp.float32)

    def zbody(i, carry):
        deg_local[pl.ds(i * 16, 16)] = zero16
        return carry

    lax.fori_loop(0, N // 16, zbody, 0)

    lo = wid * NGROUPS // NW
    hi = (wid + 1) * NGROUPS // NW

    def gbody(g, carry):
        pltpu.sync_copy(col_hbm.at[pl.ds(g * G, G)], colbuf)
        pltpu.sync_copy(ew_hbm.at[pl.ds(g * G, G)], ewbuf)

        def inner(k, carry2):
            cv = colbuf[pl.ds(k * 16, 16)]
            wv = ewbuf[pl.ds(k * 16, 16)]
            plsc.addupdate_scatter(deg_local, [cv], wv)
            return carry2

        lax.fori_loop(0, G // 16, inner, 0)
        return carry

    lax.fori_loop(lo, hi, gbody, 0)
    pltpu.sync_copy(deg_local, degp_hbm.at[wid])


@functools.partial(
    pl.kernel,
    out_type=jax.ShapeDtypeStruct((NC, N, D), jnp.float32),
    mesh=_mesh,
    scratch_types=[
        pltpu.VMEM((G,), jnp.int32),
        pltpu.VMEM((G,), jnp.int32),
        pltpu.VMEM((G,), jnp.float32),
        pltpu.VMEM((G, D), jnp.float32),
        pltpu.VMEM_SHARED((N, D), jnp.float32),
        pltpu.SemaphoreType.DMA,
    ],
    compiler_params=pltpu.CompilerParams(needs_layout_passes=False, use_tc_tiling_on_sc=False),
)
def _layer_kernel(row_hbm, col_hbm, ew_hbm, h_hbm, zeros_hbm, outp_hbm,
                  rowidx, colidx, ewbuf, rows, acc, dma_sem):
    c = lax.axis_index("c")
    s = lax.axis_index("s")
    wid = s * NC + c

    pltpu.sync_copy(zeros_hbm.at[pl.ds(s * ROWS_PER_TILE, ROWS_PER_TILE)],
                    acc.at[pl.ds(s * ROWS_PER_TILE, ROWS_PER_TILE)])
    plsc.subcore_barrier()

    lo = wid * NGROUPS // NW
    hi = (wid + 1) * NGROUPS // NW

    def gbody(g, carry):
        base = g * G
        pltpu.sync_copy(row_hbm.at[pl.ds(base, G)], rowidx)
        pltpu.sync_copy(col_hbm.at[pl.ds(base, G)], colidx)
        pltpu.sync_copy(ew_hbm.at[pl.ds(base, G)], ewbuf)
        pltpu.async_copy(h_hbm.at[rowidx], rows, dma_sem).wait()

        def scale(k, carry2):
            e0 = k * 16
            for j in range(16):
                w16 = plsc.load_gather(
                    ewbuf, [jnp.full((16,), e0 + j, jnp.int32)])
                for f in range(D // 16):
                    sl = pl.ds(f * 16, 16)
                    rows[e0 + j, sl] = rows[e0 + j, sl] * w16
            return carry2

        lax.fori_loop(0, G // 16, scale, 0)

        pltpu.sync_copy(rows, acc.at[colidx], add=True)
        return carry

    lax.fori_loop(lo, hi, gbody, 0)
    plsc.subcore_barrier()

    pltpu.sync_copy(acc.at[pl.ds(s * ROWS_PER_TILE, ROWS_PER_TILE)],
                    outp_hbm.at[c, pl.ds(s * ROWS_PER_TILE, ROWS_PER_TILE)])


def _dis_body(degp_ref, dis_ref):
    deg = jnp.sum(degp_ref[...], axis=0, keepdims=True) + 1.0
    safe = jnp.where(deg > 0, deg, 1.0)
    dis_ref[...] = jnp.where(deg > 0, lax.rsqrt(safe), 0.0)


def _dis_call(degp):
    return pl.pallas_call(
        _dis_body,
        out_shape=jax.ShapeDtypeStruct((1, N), jnp.float32),
    )(degp)


_BLK = 2000
_NBLK = N // _BLK


def _mm_body(x_ref, w_ref, disc_ref, out_ref):
    h = lax.dot_general(x_ref[...], w_ref[...], (((1,), (1,)), ((), ())),
                        preferred_element_type=jnp.float32)
    out_ref[...] = h * disc_ref[...]


def _mm_call(x, w, disc):
    return pl.pallas_call(
        _mm_body,
        grid=(_NBLK,),
        in_specs=[
            pl.BlockSpec((_BLK, D), lambda i: (i, 0)),
            pl.BlockSpec((D, D), lambda i: (0, 0)),
            pl.BlockSpec((_BLK, 1), lambda i: (i, 0)),
        ],
        out_specs=pl.BlockSpec((_BLK, D), lambda i: (i, 0)),
        out_shape=jax.ShapeDtypeStruct((N, D), jnp.float32),
    )(x, w, disc)


def _mid_body(p_ref, hp_ref, disc_ref, b_ref, w_ref, out_ref):
    srow = p_ref[0] + p_ref[1] + hp_ref[...]
    z = jnp.maximum(disc_ref[...] * srow + b_ref[...], 0.0)
    h2 = lax.dot_general(z, w_ref[...], (((1,), (1,)), ((), ())),
                         preferred_element_type=jnp.float32)
    out_ref[...] = h2 * disc_ref[...]


def _mid_call(p, hp, disc, b, w):
    return pl.pallas_call(
        _mid_body,
        grid=(_NBLK,),
        in_specs=[
            pl.BlockSpec((NC, _BLK, D), lambda i: (0, i, 0)),
            pl.BlockSpec((_BLK, D), lambda i: (i, 0)),
            pl.BlockSpec((_BLK, 1), lambda i: (i, 0)),
            pl.BlockSpec((1, D), lambda i: (0, 0)),
            pl.BlockSpec((D, D), lambda i: (0, 0)),
        ],
        out_specs=pl.BlockSpec((_BLK, D), lambda i: (i, 0)),
        out_shape=jax.ShapeDtypeStruct((N, D), jnp.float32),
    )(p, hp, disc, b, w)


def _final_body(p_ref, hp_ref, disc_ref, b_ref, out_ref):
    srow = p_ref[0] + p_ref[1] + hp_ref[...]
    out_ref[...] = jnp.maximum(disc_ref[...] * srow + b_ref[...], 0.0)


def _final_call(p, hp, disc, b):
    return pl.pallas_call(
        _final_body,
        grid=(_NBLK,),
        in_specs=[
            pl.BlockSpec((NC, _BLK, D), lambda i: (0, i, 0)),
            pl.BlockSpec((_BLK, D), lambda i: (i, 0)),
            pl.BlockSpec((_BLK, 1), lambda i: (i, 0)),
            pl.BlockSpec((1, D), lambda i: (0, 0)),
        ],
        out_specs=pl.BlockSpec((_BLK, D), lambda i: (i, 0)),
        out_shape=jax.ShapeDtypeStruct((N, D), jnp.float32),
    )(p, hp, disc, b)


def kernel(x, edge_index, edge_weights, W1, b1, W2, b2):
    row = edge_index[0]
    col = edge_index[1]
    zeros_nd = jnp.zeros((N, D), jnp.float32)

    degp = _deg_kernel(col, edge_weights)
    dis = _dis_call(degp)
    disc = dis.reshape(N, 1)

    h1p = _mm_call(x, W1, disc)
    p1 = _layer_kernel(row, col, edge_weights, h1p, zeros_nd)
    h2p = _mid_call(p1, h1p, disc, b1.reshape(1, D), W2)
    p2 = _layer_kernel(row, col, edge_weights, h2p, zeros_nd)
    return _final_call(p2, h2p, disc, b2.reshape(1, D))


# speedup vs baseline: 14.7372x; 1.2930x over previous
import functools

import jax
import jax.numpy as jnp
from jax import lax
from jax.experimental import pallas as pl
from jax.experimental.pallas import tpu as pltpu
from jax.experimental.pallas import tpu_sc as plsc

N = 10000
E = 320000
D = 128
NC = 2
NS = 16
NW = NC * NS
G = 128
NGROUPS = E // G
ROWS_PER_TILE = N // NS

_mesh = plsc.VectorSubcoreMesh(core_axis_name="c", subcore_axis_name="s")


@functools.partial(
    pl.kernel,
    out_type=jax.ShapeDtypeStruct((NW, N), jnp.float32),
    mesh=_mesh,
    scratch_types=[
        pltpu.VMEM((G,), jnp.int32),
        pltpu.VMEM((G,), jnp.float32),
        pltpu.VMEM((N,), jnp.float32),
    ],
    compiler_params=pltpu.CompilerParams(needs_layout_passes=False, use_tc_tiling_on_sc=False),
)
def _deg_kernel(col_hbm, ew_hbm, degp_hbm, colbuf, ewbuf, deg_local):
    c = lax.axis_index("c")
    s = lax.axis_index("s")
    wid = s * NC + c

    zero16 = jnp.zeros((16,), jnp.float32)

    def zbody(i, carry):
        deg_local[pl.ds(i * 16, 16)] = zero16
        return carry

    lax.fori_loop(0, N // 16, zbody, 0)

    lo = wid * NGROUPS // NW
    hi = (wid + 1) * NGROUPS // NW

    def gbody(g, carry):
        pltpu.sync_copy(col_hbm.at[pl.ds(g * G, G)], colbuf)
        pltpu.sync_copy(ew_hbm.at[pl.ds(g * G, G)], ewbuf)

        def inner(k, carry2):
            cv = colbuf[pl.ds(k * 16, 16)]
            wv = ewbuf[pl.ds(k * 16, 16)]
            plsc.addupdate_scatter(deg_local, [cv], wv)
            return carry2

        lax.fori_loop(0, G // 16, inner, 0)
        return carry

    lax.fori_loop(lo, hi, gbody, 0)
    pltpu.sync_copy(deg_local, degp_hbm.at[wid])


@functools.partial(
    pl.kernel,
    out_type=jax.ShapeDtypeStruct((NC, N, D), jnp.float32),
    mesh=_mesh,
    scratch_types=[
        pltpu.VMEM((2, G), jnp.int32),
        pltpu.VMEM((2, G), jnp.int32),
        pltpu.VMEM((2, G), jnp.float32),
        pltpu.VMEM((2, G, D), jnp.float32),
        pltpu.VMEM_SHARED((N, D), jnp.float32),
        pltpu.SemaphoreType.DMA,
        pltpu.SemaphoreType.DMA,
        pltpu.SemaphoreType.DMA,
        pltpu.SemaphoreType.DMA,
    ],
    compiler_params=pltpu.CompilerParams(needs_layout_passes=False, use_tc_tiling_on_sc=False),
)
def _layer_kernel(row_hbm, col_hbm, ew_hbm, h_hbm, zeros_hbm, outp_hbm,
                  rowidx, colidx, ewbuf, rows, acc,
                  gsem0, gsem1, ssem0, ssem1):
    c = lax.axis_index("c")
    s = lax.axis_index("s")
    wid = s * NC + c
    gsems = (gsem0, gsem1)
    ssems = (ssem0, ssem1)

    pltpu.sync_copy(zeros_hbm.at[pl.ds(s * ROWS_PER_TILE, ROWS_PER_TILE)],
                    acc.at[pl.ds(s * ROWS_PER_TILE, ROWS_PER_TILE)])
    plsc.subcore_barrier()

    lo = wid * NGROUPS // NW
    hi = (wid + 1) * NGROUPS // NW

    def fire(g, b):
        base = g * G
        pltpu.sync_copy(row_hbm.at[pl.ds(base, G)], rowidx.at[b])
        pltpu.sync_copy(col_hbm.at[pl.ds(base, G)], colidx.at[b])
        pltpu.sync_copy(ew_hbm.at[pl.ds(base, G)], ewbuf.at[b])
        pltpu.async_copy(h_hbm.at[rowidx.at[b]], rows.at[b], gsems[b])

    def wait_gather(b):
        pltpu.make_async_copy(h_hbm.at[rowidx.at[b]], rows.at[b],
                              gsems[b]).wait()

    def fire_scatter(b):
        pltpu.async_copy(rows.at[b], acc.at[colidx.at[b]], ssems[b], add=True)

    def wait_scatter(b):
        pltpu.make_async_copy(rows.at[b], acc.at[colidx.at[b]],
                              ssems[b]).wait()

    def scale(b):
        def sbody(k, carry2):
            e0 = k * 16
            for j in range(16):
                w16 = plsc.load_gather(
                    ewbuf.at[b], [jnp.full((16,), e0 + j, jnp.int32)])
                for f in range(D // 16):
                    sl = pl.ds(f * 16, 16)
                    rows[b, e0 + j, sl] = rows[b, e0 + j, sl] * w16
            return carry2

        lax.fori_loop(0, G // 16, sbody, 0)

    fire(lo, 0)
    npairs = (NGROUPS + NW - 1) // NW
    npairs = (npairs + 1) // 2

    def pbody(p, carry):
        for b in (0, 1):
            g = lo + 2 * p + b
            nxt = g + 1
            nb = 1 - b

            @pl.when(jnp.logical_and(nxt < hi, nxt - 2 >= lo))
            def _():
                wait_scatter(nb)

            @pl.when(nxt < hi)
            def _():
                fire(nxt, nb)

            @pl.when(g < hi)
            def _():
                wait_gather(b)
                scale(b)
                fire_scatter(b)
        return carry

    lax.fori_loop(0, npairs, pbody, 0)
    wait_scatter(0)
    wait_scatter(1)
    plsc.subcore_barrier()

    pltpu.sync_copy(acc.at[pl.ds(s * ROWS_PER_TILE, ROWS_PER_TILE)],
                    outp_hbm.at[c, pl.ds(s * ROWS_PER_TILE, ROWS_PER_TILE)])


def _dis_body(degp_ref, dis_ref):
    deg = jnp.sum(degp_ref[...], axis=0, keepdims=True) + 1.0
    safe = jnp.where(deg > 0, deg, 1.0)
    dis_ref[...] = jnp.where(deg > 0, lax.rsqrt(safe), 0.0)


def _dis_call(degp):
    return pl.pallas_call(
        _dis_body,
        out_shape=jax.ShapeDtypeStruct((1, N), jnp.float32),
    )(degp)


_BLK = 2000
_NBLK = N // _BLK


def _mm_body(x_ref, w_ref, disc_ref, out_ref):
    h = lax.dot_general(x_ref[...], w_ref[...], (((1,), (1,)), ((), ())),
                        preferred_element_type=jnp.float32)
    out_ref[...] = h * disc_ref[...]


def _mm_call(x, w, disc):
    return pl.pallas_call(
        _mm_body,
        grid=(_NBLK,),
        in_specs=[
            pl.BlockSpec((_BLK, D), lambda i: (i, 0)),
            pl.BlockSpec((D, D), lambda i: (0, 0)),
            pl.BlockSpec((_BLK, 1), lambda i: (i, 0)),
        ],
        out_specs=pl.BlockSpec((_BLK, D), lambda i: (i, 0)),
        out_shape=jax.ShapeDtypeStruct((N, D), jnp.float32),
    )(x, w, disc)


def _mid_body(p_ref, hp_ref, disc_ref, b_ref, w_ref, out_ref):
    srow = p_ref[0] + p_ref[1] + hp_ref[...]
    z = jnp.maximum(disc_ref[...] * srow + b_ref[...], 0.0)
    h2 = lax.dot_general(z, w_ref[...], (((1,), (1,)), ((), ())),
                         preferred_element_type=jnp.float32)
    out_ref[...] = h2 * disc_ref[...]


def _mid_call(p, hp, disc, b, w):
    return pl.pallas_call(
        _mid_body,
        grid=(_NBLK,),
        in_specs=[
            pl.BlockSpec((NC, _BLK, D), lambda i: (0, i, 0)),
            pl.BlockSpec((_BLK, D), lambda i: (i, 0)),
            pl.BlockSpec((_BLK, 1), lambda i: (i, 0)),
            pl.BlockSpec((1, D), lambda i: (0, 0)),
            pl.BlockSpec((D, D), lambda i: (0, 0)),
        ],
        out_specs=pl.BlockSpec((_BLK, D), lambda i: (i, 0)),
        out_shape=jax.ShapeDtypeStruct((N, D), jnp.float32),
    )(p, hp, disc, b, w)


def _final_body(p_ref, hp_ref, disc_ref, b_ref, out_ref):
    srow = p_ref[0] + p_ref[1] + hp_ref[...]
    out_ref[...] = jnp.maximum(disc_ref[...] * srow + b_ref[...], 0.0)


def _final_call(p, hp, disc, b):
    return pl.pallas_call(
        _final_body,
        grid=(_NBLK,),
        in_specs=[
            pl.BlockSpec((NC, _BLK, D), lambda i: (0, i, 0)),
            pl.BlockSpec((_BLK, D), lambda i: (i, 0)),
            pl.BlockSpec((_BLK, 1), lambda i: (i, 0)),
            pl.BlockSpec((1, D), lambda i: (0, 0)),
        ],
        out_specs=pl.BlockSpec((_BLK, D), lambda i: (i, 0)),
        out_shape=jax.ShapeDtypeStruct((N, D), jnp.float32),
    )(p, hp, disc, b)


def kernel(x, edge_index, edge_weights, W1, b1, W2, b2):
    row = edge_index[0]
    col = edge_index[1]
    zeros_nd = jnp.zeros((N, D), jnp.float32)

    degp = _deg_kernel(col, edge_weights)
    dis = _dis_call(degp)
    disc = dis.reshape(N, 1)

    h1p = _mm_call(x, W1, disc)
    p1 = _layer_kernel(row, col, edge_weights, h1p, zeros_nd)
    h2p = _mid_call(p1, h1p, disc, b1.reshape(1, D), W2)
    p2 = _layer_kernel(row, col, edge_weights, h2p, zeros_nd)
    return _final_call(p2, h2p, disc, b2.reshape(1, D))
